# Initial kernel scaffold; baseline (speedup 1.0000x reference)
import jax
import jax.numpy as jnp
from jax.experimental import pallas as pl


def kernel(prefix, table):
    raise NotImplementedError("write your pallas kernel here")


# speedup vs baseline: 2.3887x; 2.3887x over previous
import functools

import jax
import jax.numpy as jnp
from jax import lax
from jax.experimental import pallas as pl
from jax.experimental.pallas import tpu as pltpu
from jax.experimental.pallas import tpu_sc as plsc

EMBED = 49152
BATCH_ROWS = 2048
NC, NS = 2, 16
NW = NC * NS
RPW = BATCH_ROWS // NW
RG = 8
NG = RPW // RG
DC = 6144
NCH = EMBED // DC


def _gather_sc(idx, table):
    mesh = plsc.VectorSubcoreMesh(core_axis_name="c", subcore_axis_name="s")

    @functools.partial(
        pl.kernel,
        mesh=mesh,
        out_type=jax.ShapeDtypeStruct((BATCH_ROWS, EMBED), jnp.float32),
        scratch_types=[
            pltpu.VMEM((RPW,), jnp.int32),
            pltpu.VMEM((RG, DC), jnp.float32),
            pltpu.VMEM((RG, DC), jnp.float32),
            pltpu.SemaphoreType.DMA,
            pltpu.SemaphoreType.DMA,
            pltpu.SemaphoreType.DMA,
            pltpu.SemaphoreType.DMA,
        ],
    )
    def k(idx_hbm, table_hbm, out_hbm, idx_v, buf0, buf1,
          gsem0, gsem1, wsem0, wsem1):
        wid = lax.axis_index("s") * NC + lax.axis_index("c")
        base = wid * RPW
        pltpu.sync_copy(idx_hbm.at[pl.ds(base, RPW)], idx_v)

        bufs = (buf0, buf1)
        gsems = (gsem0, gsem1)
        wsems = (wsem0, wsem1)

        def src(g, c):
            return table_hbm.at[idx_v.at[pl.ds(g * RG, RG)],
                                pl.ds(c * DC, DC)]

        def dst(g, c):
            return out_hbm.at[pl.ds(base + g * RG, RG), pl.ds(c * DC, DC)]

        pltpu.make_async_copy(src(0, 0), buf0, gsem0).start()
        pltpu.make_async_copy(src(0, 1), buf1, gsem1).start()

        def body(g, carry):
            for c in range(NCH):
                b = c % 2
                pltpu.make_async_copy(src(g, c), bufs[b], gsems[b]).wait()
                w = pltpu.make_async_copy(bufs[b], dst(g, c), wsems[b])
                w.start()
                w.wait()
                nc = c + 2
                if nc < NCH:
                    pltpu.make_async_copy(src(g, nc), bufs[b],
                                          gsems[b]).start()
                else:
                    @pl.when(g + 1 < NG)
                    def _():
                        pltpu.make_async_copy(src(g + 1, nc - NCH), bufs[b],
                                              gsems[b]).start()
            return carry

        lax.fori_loop(0, NG, body, None)

    return k(idx, table)


def kernel(prefix, table):
    idx = prefix.reshape(-1).astype(jnp.int32)
    out = _gather_sc(idx, table)
    return out.reshape(prefix.shape[0], prefix.shape[1], EMBED)


# speedup vs baseline: 3.8537x; 1.6133x over previous
import functools

import jax
import jax.numpy as jnp
from jax import lax
from jax.experimental import pallas as pl
from jax.experimental.pallas import tpu as pltpu
from jax.experimental.pallas import tpu_sc as plsc

EMBED = 49152
BATCH_ROWS = 2048
NC, NS = 2, 16
NW = NC * NS
SPW = BATCH_ROWS // NW


def _scatter_sorted(vals, order, table):
    mesh = plsc.VectorSubcoreMesh(core_axis_name="c", subcore_axis_name="s")

    @functools.partial(
        pl.kernel,
        mesh=mesh,
        out_type=jax.ShapeDtypeStruct((BATCH_ROWS, EMBED), jnp.float32),
        scratch_types=[
            pltpu.VMEM((SPW + 16,), jnp.int32),
            pltpu.VMEM((SPW + 16,), jnp.int32),
            pltpu.VMEM((1, EMBED), jnp.float32),
        ],
    )
    def k(vals_hbm, order_hbm, table_hbm, out_hbm, vals_v, order_v, buf):
        wid = lax.axis_index("s") * NC + lax.axis_index("c")
        base = wid * SPW
        pltpu.sync_copy(vals_hbm.at[pl.ds(base, SPW)],
                        vals_v.at[pl.ds(0, SPW)])
        pltpu.sync_copy(order_hbm.at[pl.ds(base, SPW)],
                        order_v.at[pl.ds(0, SPW)])

        def slot(j, cur):
            v = vals_v[pl.ds(j, 16)][0]
            r = order_v[pl.ds(j, 16)][0]

            @pl.when(v != cur)
            def _():
                pltpu.sync_copy(table_hbm.at[pl.ds(v, 1)], buf)

            pltpu.sync_copy(buf, out_hbm.at[pl.ds(r, 1)])
            return v

        lax.fori_loop(0, SPW, slot, jnp.int32(-1))

    return k(vals, order, table)


def kernel(prefix, table):
    idx = prefix.reshape(-1).astype(jnp.int32)
    order = jnp.argsort(idx).astype(jnp.int32)
    vals = jnp.take(idx, order)
    out = _scatter_sorted(vals, order, table)
    return out.reshape(prefix.shape[0], prefix.shape[1], EMBED)


# speedup vs baseline: 3.9142x; 1.0157x over previous
import functools

import jax
import jax.numpy as jnp
from jax import lax
from jax.experimental import pallas as pl
from jax.experimental.pallas import tpu as pltpu
from jax.experimental.pallas import tpu_sc as plsc

EMBED = 49152
BATCH_ROWS = 2048
NC, NS = 2, 16
NW = NC * NS
SPW = BATCH_ROWS // NW


def _scatter_sorted(idx, order, table):
    mesh = plsc.VectorSubcoreMesh(core_axis_name="c", subcore_axis_name="s")

    @functools.partial(
        pl.kernel,
        mesh=mesh,
        out_type=jax.ShapeDtypeStruct((BATCH_ROWS, EMBED), jnp.float32),
        scratch_types=[
            pltpu.VMEM((BATCH_ROWS + 16,), jnp.int32),
            pltpu.VMEM((SPW + 16,), jnp.int32),
            pltpu.VMEM((2, EMBED), jnp.float32),
            pltpu.SemaphoreType.DMA,
        ],
    )
    def k(idx_hbm, order_hbm, table_hbm, out_hbm, idx_v, order_v, buf, psem):
        wid = lax.axis_index("s") * NC + lax.axis_index("c")
        base = wid * SPW
        pltpu.sync_copy(idx_hbm, idx_v.at[pl.ds(0, BATCH_ROWS)])
        pltpu.sync_copy(order_hbm.at[pl.ds(base, SPW)],
                        order_v.at[pl.ds(0, SPW)])

        def val_at(j):
            r = order_v[pl.ds(j, 16)][0]
            rc = lax.min(lax.max(r, jnp.int32(0)),
                         jnp.int32(BATCH_ROWS - 1))
            return r, idx_v[pl.ds(rc, 16)][0]

        def slot(j, carry):
            cur, p, pending = carry
            r, v = val_at(j)

            new_run = v != cur

            @pl.when(new_run & (pending == 1))
            def _():
                pltpu.make_async_copy(
                    table_hbm.at[pl.ds(v, 1)],
                    buf.at[pl.ds(1 - p, 1)], psem).wait()

            @pl.when(new_run & (pending == 0))
            def _():
                pltpu.sync_copy(table_hbm.at[pl.ds(v, 1)],
                                buf.at[pl.ds(1 - p, 1)])

            p = jnp.where(new_run, 1 - p, p)
            pltpu.sync_copy(buf.at[pl.ds(p, 1)], out_hbm.at[pl.ds(r, 1)])

            rn, vn = val_at(j + 1)
            do_pre = (j < SPW - 1) & (vn != v)

            @pl.when(do_pre)
            def _():
                pltpu.make_async_copy(
                    table_hbm.at[pl.ds(vn, 1)],
                    buf.at[pl.ds(1 - p, 1)], psem).start()

            return (v, p, jnp.where(do_pre, 1, 0).astype(jnp.int32))

        lax.fori_loop(0, SPW, slot,
                      (jnp.int32(-1), jnp.int32(1), jnp.int32(0)))

    return k(idx, order, table)


def kernel(prefix, table):
    idx = prefix.reshape(-1).astype(jnp.int32)
    order = jnp.argsort(idx).astype(jnp.int32)
    out = _scatter_sorted(idx, order, table)
    return out.reshape(prefix.shape[0], prefix.shape[1], EMBED)


# speedup vs baseline: 3.9422x; 1.0072x over previous
import functools

import jax
import jax.numpy as jnp
from jax import lax
from jax.experimental import pallas as pl
from jax.experimental.pallas import tpu as pltpu
from jax.experimental.pallas import tpu_sc as plsc

EMBED = 49152
BATCH_ROWS = 2048
NC, NS = 2, 16
NW = NC * NS
SPW = BATCH_ROWS // NW


def _scatter_sorted(idx, order, table):
    mesh = plsc.VectorSubcoreMesh(core_axis_name="c", subcore_axis_name="s")

    @functools.partial(
        pl.kernel,
        mesh=mesh,
        out_type=jax.ShapeDtypeStruct((BATCH_ROWS, EMBED), jnp.float32),
        scratch_types=[
            pltpu.VMEM((BATCH_ROWS + 16,), jnp.int32),
            pltpu.VMEM((SPW + 16,), jnp.int32),
            pltpu.VMEM((2, EMBED), jnp.float32),
            pltpu.SemaphoreType.DMA,
            pltpu.SemaphoreType.DMA,
            pltpu.SemaphoreType.DMA,
        ],
    )
    def k(idx_hbm, order_hbm, table_hbm, out_hbm, idx_v, order_v, buf,
          psem, wsem0, wsem1):
        wid = lax.axis_index("s") * NC + lax.axis_index("c")
        base = wid * SPW
        pltpu.sync_copy(idx_hbm, idx_v.at[pl.ds(0, BATCH_ROWS)])
        pltpu.sync_copy(order_hbm.at[pl.ds(base, SPW)],
                        order_v.at[pl.ds(0, SPW)])

        def val_at(j):
            r = order_v[pl.ds(j, 16)][0]
            rc = lax.min(lax.max(r, jnp.int32(0)),
                         jnp.int32(BATCH_ROWS - 1))
            return r, idx_v[pl.ds(rc, 16)][0]

        def wait_writes(sem):
            def w(i, c):
                pltpu.make_async_copy(buf.at[pl.ds(0, 1)],
                                      out_hbm.at[pl.ds(0, 1)], sem).wait()
                return c
            return w

        def slot(j, carry):
            cur, p, pending, n0, n1 = carry
            r, v = val_at(j)

            new_run = v != cur

            @pl.when(new_run & (pending == 1))
            def _():
                pltpu.make_async_copy(
                    table_hbm.at[pl.ds(v, 1)],
                    buf.at[pl.ds(1 - p, 1)], psem).wait()

            @pl.when(new_run & (pending == 0))
            def _():
                pltpu.sync_copy(table_hbm.at[pl.ds(v, 1)],
                                buf.at[pl.ds(1 - p, 1)])

            pp = jnp.where(new_run, 1 - p, p)

            @pl.when(pp == 0)
            def _():
                pltpu.make_async_copy(buf.at[pl.ds(0, 1)],
                                      out_hbm.at[pl.ds(r, 1)], wsem0).start()

            @pl.when(pp == 1)
            def _():
                pltpu.make_async_copy(buf.at[pl.ds(1, 1)],
                                      out_hbm.at[pl.ds(r, 1)], wsem1).start()

            n0 = n0 + jnp.where(pp == 0, 1, 0).astype(jnp.int32)
            n1 = n1 + jnp.where(pp == 1, 1, 0).astype(jnp.int32)

            rn, vn = val_at(j + 1)
            do_pre = (j < SPW - 1) & (vn != v)

            @pl.when(do_pre & (pp == 1))
            def _():
                lax.fori_loop(0, n0, wait_writes(wsem0), jnp.int32(0))
                pltpu.make_async_copy(table_hbm.at[pl.ds(vn, 1)],
                                      buf.at[pl.ds(0, 1)], psem).start()

            @pl.when(do_pre & (pp == 0))
            def _():
                lax.fori_loop(0, n1, wait_writes(wsem1), jnp.int32(0))
                pltpu.make_async_copy(table_hbm.at[pl.ds(vn, 1)],
                                      buf.at[pl.ds(1, 1)], psem).start()

            n0 = jnp.where(do_pre & (pp == 1), 0, n0).astype(jnp.int32)
            n1 = jnp.where(do_pre & (pp == 0), 0, n1).astype(jnp.int32)

            return (v, pp, jnp.where(do_pre, 1, 0).astype(jnp.int32),
                    n0, n1)

        _, _, _, n0, n1 = lax.fori_loop(
            0, SPW, slot,
            (jnp.int32(-1), jnp.int32(1), jnp.int32(0),
             jnp.int32(0), jnp.int32(0)))
        lax.fori_loop(0, n0, wait_writes(wsem0), jnp.int32(0))
        lax.fori_loop(0, n1, wait_writes(wsem1), jnp.int32(0))

    return k(idx, order, table)


def kernel(prefix, table):
    idx = prefix.reshape(-1).astype(jnp.int32)
    order = jnp.argsort(idx).astype(jnp.int32)
    out = _scatter_sorted(idx, order, table)
    return out.reshape(prefix.shape[0], prefix.shape[1], EMBED)
